# Initial kernel scaffold; baseline (speedup 1.0000x reference)
import jax
import jax.numpy as jnp
from jax.experimental import pallas as pl


def kernel(x, adj, params):
    raise NotImplementedError("write your pallas kernel here")


# speedup vs baseline: 1.1634x; 1.1634x over previous
import functools

import jax
import jax.numpy as jnp
from jax.experimental import pallas as pl
from jax.experimental.pallas import tpu as pltpu

F32 = jnp.float32
BF16 = jnp.bfloat16
SIGMA = 0.5
NPAD = 128
N_REAL = 10


def _split_f32(a):
    hi = a.astype(BF16)
    lo = (a - hi.astype(F32)).astype(BF16)
    return hi, lo


def _mm3(a, w_hi, w_lo):
    a_hi = a.astype(BF16)
    a_lo = (a - a_hi.astype(F32)).astype(BF16)
    d = functools.partial(jnp.dot, preferred_element_type=F32)
    return d(a_hi, w_hi) + (d(a_lo, w_hi) + d(a_hi, w_lo))


def _ae_kernel(x_ref,
               e1h, e1l, e1b, e2h, e2l, e2b, e3h, e3l, e3b,
               zlh, zll, zlb, d1h, d1l, d1b, d2h, d2l, d2b,
               d3h, d3l, d3b, xbh, xbl, xbb, g1h, cT,
               xbar_o, zpad_o, qpad_o, h1_o, h2_o, h3_o, u1_o):
    x = x_ref[...]
    h1 = jax.nn.relu(_mm3(x, e1h[...], e1l[...]) + e1b[...])
    h2 = jax.nn.relu(_mm3(h1, e2h[...], e2l[...]) + e2b[...])
    h3 = jax.nn.relu(_mm3(h2, e3h[...], e3l[...]) + e3b[...])
    z = _mm3(h3, zlh[...], zll[...]) + zlb[...]
    d1 = jax.nn.relu(_mm3(z, d1h[...], d1l[...]) + d1b[...])
    d2 = jax.nn.relu(_mm3(d1, d2h[...], d2l[...]) + d2b[...])
    d3 = jax.nn.relu(_mm3(d2, d3h[...], d3l[...]) + d3b[...])
    x_bar = _mm3(d3, xbh[...], xbl[...]) + xbb[...]

    xbar_o[...] = x_bar
    zpad_o[...] = z
    h1_o[...] = h1.astype(BF16)
    h2_o[...] = h2.astype(BF16)
    h3_o[...] = h3.astype(BF16)
    u1_o[...] = jnp.dot(x.astype(BF16), g1h[...], preferred_element_type=F32).astype(BF16)

    cT = cT[...]
    zn = jnp.sum(z * z, axis=1, keepdims=True)
    cn = jnp.sum(cT * cT, axis=0).reshape(1, NPAD)
    cross = jnp.dot(z, cT, preferred_element_type=F32)
    d = zn + cn - 2.0 * cross
    mask = jax.lax.broadcasted_iota(jnp.int32, d.shape, 1) < N_REAL
    qv = jnp.where(mask, 1.0 / (1.0 + d), 0.0)
    qpad_o[...] = qv / jnp.sum(qv, axis=1, keepdims=True)


def _run_ae(x, p, bm):
    M, d_in = x.shape
    n1, n2, n3 = 500, 500, 2000

    def wsplit(name):
        return _split_f32(p[name + '_w'])

    def bias(name, width=None):
        b = p[name + '_b']
        if width is not None:
            b = jnp.pad(b, (0, width - b.shape[0]))
        return b.reshape(1, -1)

    e1h, e1l = wsplit('enc1')
    e2h, e2l = wsplit('enc2')
    e3h, e3l = wsplit('enc3')
    zlw = jnp.pad(p['zl_w'], ((0, 0), (0, NPAD - N_REAL)))
    zlh, zll = _split_f32(zlw)
    d1w = jnp.pad(p['dec1_w'], ((0, NPAD - N_REAL), (0, 0)))
    d1h, d1l = _split_f32(d1w)
    d2h, d2l = wsplit('dec2')
    d3h, d3l = wsplit('dec3')
    xbh, xbl = wsplit('xbar')
    g1h = p['gnn1_w'].astype(BF16)
    cT = jnp.pad(p['cluster'].T, ((0, NPAD - N_REAL), (0, NPAD - N_REAL)))

    operands = [x,
                e1h, e1l, bias('enc1'), e2h, e2l, bias('enc2'),
                e3h, e3l, bias('enc3'), zlh, zll, bias('zl', NPAD),
                d1h, d1l, bias('dec1'), d2h, d2l, bias('dec2'),
                d3h, d3l, bias('dec3'), xbh, xbl, bias('xbar'),
                g1h, cT]

    def full_spec(a):
        return pl.BlockSpec(a.shape, lambda i: (0,) * a.ndim)

    in_specs = [pl.BlockSpec((bm, d_in), lambda i: (i, 0))]
    in_specs += [full_spec(a) for a in operands[1:]]

    out_shape = [
        jax.ShapeDtypeStruct((M, d_in), F32),
        jax.ShapeDtypeStruct((M, NPAD), F32),
        jax.ShapeDtypeStruct((M, NPAD), F32),
        jax.ShapeDtypeStruct((M, n1), BF16),
        jax.ShapeDtypeStruct((M, n2), BF16),
        jax.ShapeDtypeStruct((M, n3), BF16),
        jax.ShapeDtypeStruct((M, n1), BF16),
    ]
    out_specs = [pl.BlockSpec((bm, s.shape[1]), lambda i: (i, 0)) for s in out_shape]

    return pl.pallas_call(
        _ae_kernel,
        grid=(M // bm,),
        in_specs=in_specs,
        out_specs=out_specs,
        out_shape=out_shape,
        compiler_params=pltpu.CompilerParams(
            dimension_semantics=("parallel",)),
    )(*operands)


def _gnn_kernel(adj_ref, u_ref, tra_ref, w_ref, out_ref):
    acc = jnp.dot(adj_ref[...], u_ref[...], preferred_element_type=F32)
    h = jax.nn.relu(acc)
    mix = (1.0 - SIGMA) * h + SIGMA * tra_ref[...].astype(F32)
    out_ref[...] = jnp.dot(mix.astype(BF16), w_ref[...],
                           preferred_element_type=F32).astype(BF16)


def _gnn_layer(adj_b, u, tra, w, *, bm):
    M, K = adj_b.shape
    n = u.shape[1]
    n_out = w.shape[1]
    return pl.pallas_call(
        _gnn_kernel,
        grid=(M // bm,),
        in_specs=[
            pl.BlockSpec((bm, K), lambda i: (i, 0)),
            pl.BlockSpec((K, n), lambda i: (0, 0)),
            pl.BlockSpec((bm, n), lambda i: (i, 0)),
            pl.BlockSpec((n, n_out), lambda i: (0, 0)),
        ],
        out_specs=pl.BlockSpec((bm, n_out), lambda i: (i, 0)),
        out_shape=jax.ShapeDtypeStruct((M, n_out), BF16),
        compiler_params=pltpu.CompilerParams(
            dimension_semantics=("arbitrary",)),
    )(adj_b, u, tra, w)


def _gnn_last_kernel(adj_ref, u_ref, out_ref):
    acc = jnp.dot(adj_ref[...], u_ref[...], preferred_element_type=F32)
    mask = jax.lax.broadcasted_iota(jnp.int32, acc.shape, 1) < N_REAL
    logits = jnp.where(mask, acc, -1e30)
    m = jnp.max(logits, axis=1, keepdims=True)
    e = jnp.exp(logits - m)
    out_ref[...] = e / jnp.sum(e, axis=1, keepdims=True)


def _gnn_last(adj_b, u, *, bm):
    M, K = adj_b.shape
    n = u.shape[1]
    return pl.pallas_call(
        _gnn_last_kernel,
        grid=(M // bm,),
        in_specs=[
            pl.BlockSpec((bm, K), lambda i: (i, 0)),
            pl.BlockSpec((K, n), lambda i: (0, 0)),
        ],
        out_specs=pl.BlockSpec((bm, n), lambda i: (i, 0)),
        out_shape=jax.ShapeDtypeStruct((M, n), F32),
        compiler_params=pltpu.CompilerParams(
            dimension_semantics=("arbitrary",)),
    )(adj_b, u)


def kernel(x, adj, params):
    p = params
    adj_b = adj.astype(BF16)

    x_bar, z_pad, q_pad, h1, h2, h3, u1 = _run_ae(x, p, bm=400)

    g4 = jnp.pad(p['gnn4_w'], ((0, 0), (0, NPAD - N_REAL))).astype(BF16)
    g5 = jnp.pad(p['gnn5_w'], ((0, NPAD - N_REAL), (0, NPAD - N_REAL))).astype(BF16)
    z_b = z_pad.astype(BF16)

    u2 = _gnn_layer(adj_b, u1, h1, p['gnn2_w'].astype(BF16), bm=400)
    u3 = _gnn_layer(adj_b, u2, h2, p['gnn3_w'].astype(BF16), bm=400)
    u4 = _gnn_layer(adj_b, u3, h3, g4, bm=200)
    u5 = _gnn_layer(adj_b, u4, z_b, g5, bm=400)
    pred_pad = _gnn_last(adj_b, u5, bm=400)

    q = q_pad[:, :N_REAL]
    predict = pred_pad[:, :N_REAL]
    z = z_pad[:, :N_REAL]
    return (x_bar, q, predict, z)


# speedup vs baseline: 1.2597x; 1.0827x over previous
import functools

import jax
import jax.numpy as jnp
from jax.experimental import pallas as pl
from jax.experimental.pallas import tpu as pltpu

F32 = jnp.float32
BF16 = jnp.bfloat16
SIGMA = 0.5
NPAD = 128
N_REAL = 10


def _split_f32(a):
    hi = a.astype(BF16)
    lo = (a - hi.astype(F32)).astype(BF16)
    return hi, lo


def _mm3(a, w_hi, w_lo):
    a_hi = a.astype(BF16)
    a_lo = (a - a_hi.astype(F32)).astype(BF16)
    d = functools.partial(jnp.dot, preferred_element_type=F32)
    return d(a_hi, w_hi) + (d(a_lo, w_hi) + d(a_hi, w_lo))


def _ae_kernel(x_ref, adj_ref,
               e1h, e1l, e1b, e2h, e2l, e2b, e3h, e3l, e3b,
               zlh, zll, zlb, d1h, d1l, d1b, d2h, d2l, d2b,
               d3h, d3l, d3b, xbh, xbl, xbb, g1h, cT,
               xbar_o, zpad_o, qpad_o, h1_o, h2_o, h3_o, u1_o, adjb_o):
    adjb_o[...] = adj_ref[...].astype(BF16)
    x = x_ref[...]
    h1 = jax.nn.relu(_mm3(x, e1h[...], e1l[...]) + e1b[...])
    h2 = jax.nn.relu(_mm3(h1, e2h[...], e2l[...]) + e2b[...])
    h3 = jax.nn.relu(_mm3(h2, e3h[...], e3l[...]) + e3b[...])
    z = _mm3(h3, zlh[...], zll[...]) + zlb[...]
    d1 = jax.nn.relu(_mm3(z, d1h[...], d1l[...]) + d1b[...])
    d2 = jax.nn.relu(_mm3(d1, d2h[...], d2l[...]) + d2b[...])
    d3 = jax.nn.relu(_mm3(d2, d3h[...], d3l[...]) + d3b[...])
    x_bar = _mm3(d3, xbh[...], xbl[...]) + xbb[...]

    xbar_o[...] = x_bar
    zpad_o[...] = z
    h1_o[...] = h1.astype(BF16)
    h2_o[...] = h2.astype(BF16)
    h3_o[...] = h3.astype(BF16)
    u1_o[...] = jnp.dot(x.astype(BF16), g1h[...], preferred_element_type=F32).astype(BF16)

    cT = cT[...]
    zn = jnp.sum(z * z, axis=1, keepdims=True)
    cn = jnp.sum(cT * cT, axis=0).reshape(1, NPAD)
    cross = jnp.dot(z, cT, preferred_element_type=F32)
    d = zn + cn - 2.0 * cross
    mask = jax.lax.broadcasted_iota(jnp.int32, d.shape, 1) < N_REAL
    qv = jnp.where(mask, 1.0 / (1.0 + d), 0.0)
    qpad_o[...] = qv / jnp.sum(qv, axis=1, keepdims=True)


def _run_ae(x, adj, p, bm):
    M, d_in = x.shape
    K = adj.shape[1]
    n1, n2, n3 = 500, 500, 2000

    def wsplit(name):
        return _split_f32(p[name + '_w'])

    def bias(name, width=None):
        b = p[name + '_b']
        if width is not None:
            b = jnp.pad(b, (0, width - b.shape[0]))
        return b.reshape(1, -1)

    e1h, e1l = wsplit('enc1')
    e2h, e2l = wsplit('enc2')
    e3h, e3l = wsplit('enc3')
    zlw = jnp.pad(p['zl_w'], ((0, 0), (0, NPAD - N_REAL)))
    zlh, zll = _split_f32(zlw)
    d1w = jnp.pad(p['dec1_w'], ((0, NPAD - N_REAL), (0, 0)))
    d1h, d1l = _split_f32(d1w)
    d2h, d2l = wsplit('dec2')
    d3h, d3l = wsplit('dec3')
    xbh, xbl = wsplit('xbar')
    g1h = p['gnn1_w'].astype(BF16)
    cT = jnp.pad(p['cluster'].T, ((0, NPAD - N_REAL), (0, NPAD - N_REAL)))

    operands = [x, adj,
                e1h, e1l, bias('enc1'), e2h, e2l, bias('enc2'),
                e3h, e3l, bias('enc3'), zlh, zll, bias('zl', NPAD),
                d1h, d1l, bias('dec1'), d2h, d2l, bias('dec2'),
                d3h, d3l, bias('dec3'), xbh, xbl, bias('xbar'),
                g1h, cT]

    def full_spec(a):
        return pl.BlockSpec(a.shape, lambda i: (0,) * a.ndim)

    in_specs = [pl.BlockSpec((bm, d_in), lambda i: (i, 0)),
                pl.BlockSpec((bm, K), lambda i: (i, 0))]
    in_specs += [full_spec(a) for a in operands[2:]]

    out_shape = [
        jax.ShapeDtypeStruct((M, d_in), F32),
        jax.ShapeDtypeStruct((M, NPAD), F32),
        jax.ShapeDtypeStruct((M, NPAD), F32),
        jax.ShapeDtypeStruct((M, n1), BF16),
        jax.ShapeDtypeStruct((M, n2), BF16),
        jax.ShapeDtypeStruct((M, n3), BF16),
        jax.ShapeDtypeStruct((M, n1), BF16),
        jax.ShapeDtypeStruct((M, K), BF16),
    ]
    out_specs = [pl.BlockSpec((bm, s.shape[1]), lambda i: (i, 0)) for s in out_shape]

    return pl.pallas_call(
        _ae_kernel,
        grid=(M // bm,),
        in_specs=in_specs,
        out_specs=out_specs,
        out_shape=out_shape,
        compiler_params=pltpu.CompilerParams(
            dimension_semantics=("parallel",)),
    )(*operands)


def _gnn_kernel(adj_ref, u_ref, tra_ref, w_ref, out_ref):
    acc = jnp.dot(adj_ref[...], u_ref[...], preferred_element_type=F32)
    h = jax.nn.relu(acc)
    mix = (1.0 - SIGMA) * h + SIGMA * tra_ref[...].astype(F32)
    out_ref[...] = jnp.dot(mix.astype(BF16), w_ref[...],
                           preferred_element_type=F32).astype(BF16)


def _gnn_layer(adj_b, u, tra, w, *, bm):
    M, K = adj_b.shape
    n = u.shape[1]
    n_out = w.shape[1]
    return pl.pallas_call(
        _gnn_kernel,
        grid=(pl.cdiv(M, bm),),
        in_specs=[
            pl.BlockSpec((bm, K), lambda i: (i, 0)),
            pl.BlockSpec((K, n), lambda i: (0, 0)),
            pl.BlockSpec((bm, n), lambda i: (i, 0)),
            pl.BlockSpec((n, n_out), lambda i: (0, 0)),
        ],
        out_specs=pl.BlockSpec((bm, n_out), lambda i: (i, 0)),
        out_shape=jax.ShapeDtypeStruct((M, n_out), BF16),
        compiler_params=pltpu.CompilerParams(
            dimension_semantics=("arbitrary",)),
    )(adj_b, u, tra, w)


def _gnn_last_kernel(adj_ref, u_ref, out_ref):
    acc = jnp.dot(adj_ref[...], u_ref[...], preferred_element_type=F32)
    mask = jax.lax.broadcasted_iota(jnp.int32, acc.shape, 1) < N_REAL
    logits = jnp.where(mask, acc, -1e30)
    m = jnp.max(logits, axis=1, keepdims=True)
    e = jnp.exp(logits - m)
    out_ref[...] = e / jnp.sum(e, axis=1, keepdims=True)


def _gnn_last(adj_b, u, *, bm):
    M, K = adj_b.shape
    n = u.shape[1]
    return pl.pallas_call(
        _gnn_last_kernel,
        grid=(pl.cdiv(M, bm),),
        in_specs=[
            pl.BlockSpec((bm, K), lambda i: (i, 0)),
            pl.BlockSpec((K, n), lambda i: (0, 0)),
        ],
        out_specs=pl.BlockSpec((bm, n), lambda i: (i, 0)),
        out_shape=jax.ShapeDtypeStruct((M, n), F32),
        compiler_params=pltpu.CompilerParams(
            dimension_semantics=("arbitrary",)),
    )(adj_b, u)


def kernel(x, adj, params):
    p = params

    x_bar, z_pad, q_pad, h1, h2, h3, u1, adj_b = _run_ae(x, adj, p, bm=200)

    g4 = jnp.pad(p['gnn4_w'], ((0, 0), (0, NPAD - N_REAL))).astype(BF16)
    g5 = jnp.pad(p['gnn5_w'], ((0, NPAD - N_REAL), (0, NPAD - N_REAL))).astype(BF16)
    z_b = z_pad.astype(BF16)

    u2 = _gnn_layer(adj_b, u1, h1, p['gnn2_w'].astype(BF16), bm=800)
    u3 = _gnn_layer(adj_b, u2, h2, p['gnn3_w'].astype(BF16), bm=800)
    u4 = _gnn_layer(adj_b, u3, h3, g4, bm=200)
    u5 = _gnn_layer(adj_b, u4, z_b, g5, bm=1000)
    pred_pad = _gnn_last(adj_b, u5, bm=1000)

    q = q_pad[:, :N_REAL]
    predict = pred_pad[:, :N_REAL]
    z = z_pad[:, :N_REAL]
    return (x_bar, q, predict, z)
